# Initial kernel scaffold; baseline (speedup 1.0000x reference)
import jax
import jax.numpy as jnp
from jax.experimental import pallas as pl


def kernel(x1, x2, w_up, b_up, w_med, b_med, w_c, b_c):
    raise NotImplementedError("write your pallas kernel here")


# speedup vs baseline: 2.9339x; 2.9339x over previous
import functools

import jax
import jax.numpy as jnp
from jax.experimental import pallas as pl
from jax.experimental.pallas import tpu as pltpu


def _fused_body(R, W2, Cf, nt, x1_ref, x2m_ref, x2p_ref, x2n_ref,
                ww_ref, wm_ref, wc_ref, s_ref, bm_ref, bt_ref,
                out_ref, win_ref):
    t = pl.program_id(1)
    TP = R * W2
    zrow = jnp.zeros((Cf, W2), jnp.bfloat16)

    win_ref[:, 0:W2] = zrow
    win_ref[:, W2:2 * W2] = x2p_ref[0].astype(jnp.bfloat16)
    win_ref[:, 2 * W2:2 * W2 + TP] = x2m_ref[0].astype(jnp.bfloat16)
    win_ref[:, 2 * W2 + TP:3 * W2 + TP] = x2n_ref[0].astype(jnp.bfloat16)
    win_ref[:, 3 * W2 + TP:4 * W2 + TP] = zrow

    @pl.when(t == 0)
    def _():
        win_ref[:, W2:2 * W2] = zrow

    @pl.when(t == nt - 1)
    def _():
        win_ref[:, 2 * W2 + TP:3 * W2 + TP] = zrow

    lane = jax.lax.broadcasted_iota(jnp.int32, (1, TP), 1) % W2
    mneg = (lane != 0).astype(jnp.bfloat16)
    mpos = (lane != W2 - 1).astype(jnp.bfloat16)

    acc = jnp.zeros((Cf, TP), jnp.float32)
    for ky in range(3):
        for kx in range(3):
            s = W2 + ky * W2 + (kx - 1)
            opnd = win_ref[:, s:s + TP]
            if kx == 0:
                opnd = opnd * mneg
            elif kx == 2:
                opnd = opnd * mpos
            acc = acc + jnp.dot(wm_ref[ky * 3 + kx], opnd,
                                preferred_element_type=jnp.float32)
    med = acc + bm_ref[...]
    med = jnp.maximum(med, 0.2 * med).astype(jnp.bfloat16)
    res = jnp.dot(wc_ref[...], med,
                  preferred_element_type=jnp.float32) + bt_ref[...]

    up4 = jnp.dot(ww_ref[...], x1_ref[0].astype(jnp.bfloat16),
                  preferred_element_type=jnp.float32)
    W = W2 // 2
    rows = []
    for r in range(R):
        dy, h = r % 2, r // 2
        a = up4[(2 * dy) * Cf:(2 * dy + 1) * Cf, h * W:(h + 1) * W]
        b = up4[(2 * dy + 1) * Cf:(2 * dy + 2) * Cf, h * W:(h + 1) * W]
        cat = jnp.concatenate([a, b], axis=1).astype(jnp.bfloat16)
        rows.append(jnp.dot(cat, s_ref[...],
                            preferred_element_type=jnp.float32))
    out_ref[0] = res + jnp.concatenate(rows, axis=1)


def kernel(x1, x2, w_up, b_up, w_med, b_med, w_c, b_c):
    B, Cin, H, W = x1.shape
    _, Cf, H2, W2 = x2.shape
    R = next(r for r in (16, 8, 4, 2) if H2 % r == 0)
    nt = H2 // R
    TP = R * W2

    wc2 = w_c[:, :, 0, 0]
    wc_med = wc2[:, :Cf].astype(jnp.bfloat16)
    wc_up = wc2[:, Cf:]
    ww = jnp.einsum('oc,kcyx->yxok', wc_up, w_up).reshape(4 * Cf, Cin)
    ww = ww.astype(jnp.bfloat16)
    wm = jnp.transpose(w_med, (2, 3, 0, 1)).reshape(9, Cf, Cf)
    wm = wm.astype(jnp.bfloat16)
    bt = (b_c + wc_up @ b_up).reshape(Cf, 1)
    bm = b_med.reshape(Cf, 1)
    wh = jnp.arange(W2 // 2)
    s_mat = (jnp.zeros((W2, W2), jnp.bfloat16)
             .at[wh, 2 * wh].set(1)
             .at[W2 // 2 + wh, 2 * wh + 1].set(1))

    x1f = x1.reshape(B, Cin, H * W)
    x2f = x2.reshape(B, Cf, H2 * W2)

    out = pl.pallas_call(
        functools.partial(_fused_body, R, W2, Cf, nt),
        out_shape=jax.ShapeDtypeStruct((B, Cf, H2 * W2), jnp.float32),
        grid=(B, nt),
        in_specs=[
            pl.BlockSpec((1, Cin, (R // 2) * W), lambda b, t: (b, 0, t)),
            pl.BlockSpec((1, Cf, TP), lambda b, t: (b, 0, t)),
            pl.BlockSpec((1, Cf, W2),
                         lambda b, t: (b, 0, jnp.maximum(R * t - 1, 0))),
            pl.BlockSpec((1, Cf, W2),
                         lambda b, t: (b, 0, jnp.minimum(R * (t + 1), H2 - 1))),
            pl.BlockSpec((4 * Cf, Cin), lambda b, t: (0, 0)),
            pl.BlockSpec((9, Cf, Cf), lambda b, t: (0, 0, 0)),
            pl.BlockSpec((Cf, Cf), lambda b, t: (0, 0)),
            pl.BlockSpec((W2, W2), lambda b, t: (0, 0)),
            pl.BlockSpec((Cf, 1), lambda b, t: (0, 0)),
            pl.BlockSpec((Cf, 1), lambda b, t: (0, 0)),
        ],
        out_specs=pl.BlockSpec((1, Cf, TP), lambda b, t: (b, 0, t)),
        scratch_shapes=[pltpu.VMEM((Cf, TP + 4 * W2), jnp.bfloat16)],
        compiler_params=pltpu.CompilerParams(
            dimension_semantics=("parallel", "parallel"),
            vmem_limit_bytes=64 * 1024 * 1024),
    )(x1f, x2f, x2f, x2f, ww, wm, wc_med, s_mat, bm, bt)
    return out.reshape(B, Cf, H2, W2)


# speedup vs baseline: 3.2410x; 1.1047x over previous
import functools

import jax
import jax.numpy as jnp
from jax.experimental import pallas as pl
from jax.experimental.pallas import tpu as pltpu


def _fused_body(R, W2, Cf, nt, x1_ref, x2m_ref, x2p_ref, x2n_ref,
                ww_ref, wm_ref, wc_ref, s_ref, bm_ref, bt_ref,
                out_ref, win_ref):
    t = pl.program_id(1)
    TP = R * W2
    zrow = jnp.zeros((Cf, W2), jnp.bfloat16)

    win_ref[:, 0:W2] = zrow
    win_ref[:, W2:2 * W2] = x2p_ref[0].astype(jnp.bfloat16)
    win_ref[:, 2 * W2:2 * W2 + TP] = x2m_ref[0].astype(jnp.bfloat16)
    win_ref[:, 2 * W2 + TP:3 * W2 + TP] = x2n_ref[0].astype(jnp.bfloat16)
    win_ref[:, 3 * W2 + TP:4 * W2 + TP] = zrow

    @pl.when(t == 0)
    def _():
        win_ref[:, W2:2 * W2] = zrow

    @pl.when(t == nt - 1)
    def _():
        win_ref[:, 2 * W2 + TP:3 * W2 + TP] = zrow

    lane = jax.lax.broadcasted_iota(jnp.int32, (1, TP), 1) % W2
    mneg = (lane != 0).astype(jnp.bfloat16)
    mpos = (lane != W2 - 1).astype(jnp.bfloat16)

    acc = jnp.zeros((Cf, TP), jnp.float32)
    for ky in range(3):
        for kx in range(3):
            s = W2 + ky * W2 + (kx - 1)
            opnd = win_ref[:, s:s + TP]
            if kx == 0:
                opnd = opnd * mneg
            elif kx == 2:
                opnd = opnd * mpos
            acc = acc + jnp.dot(wm_ref[ky * 3 + kx], opnd,
                                preferred_element_type=jnp.float32)
    med = acc + bm_ref[...]
    med = jnp.maximum(med, 0.2 * med).astype(jnp.bfloat16)
    res = jnp.dot(wc_ref[...], med,
                  preferred_element_type=jnp.float32) + bt_ref[...]

    up4 = jnp.dot(ww_ref[...], x1_ref[0].astype(jnp.bfloat16),
                  preferred_element_type=jnp.float32)
    W = W2 // 2
    rows = []
    for r in range(R):
        dy, h = r % 2, r // 2
        a = up4[(2 * dy) * Cf:(2 * dy + 1) * Cf, h * W:(h + 1) * W]
        b = up4[(2 * dy + 1) * Cf:(2 * dy + 2) * Cf, h * W:(h + 1) * W]
        cat = jnp.concatenate([a, b], axis=1).astype(jnp.bfloat16)
        rows.append(jnp.dot(cat, s_ref[...],
                            preferred_element_type=jnp.float32))
    out_ref[0] = res + jnp.concatenate(rows, axis=1)


def kernel(x1, x2, w_up, b_up, w_med, b_med, w_c, b_c):
    B, Cin, H, W = x1.shape
    _, Cf, H2, W2 = x2.shape
    R = next(r for r in (16, 8, 4, 2) if H2 % r == 0)
    nt = H2 // R
    TP = R * W2

    wc2 = w_c[:, :, 0, 0]
    wc_med = wc2[:, :Cf].astype(jnp.bfloat16)
    wc_up = wc2[:, Cf:]
    ww = jnp.einsum('oc,kcyx->yxok', wc_up, w_up).reshape(4 * Cf, Cin)
    ww = ww.astype(jnp.bfloat16)
    wm = jnp.transpose(w_med, (2, 3, 0, 1)).reshape(9, Cf, Cf)
    wm = wm.astype(jnp.bfloat16)
    bt = (b_c + wc_up @ b_up).reshape(Cf, 1)
    bm = b_med.reshape(Cf, 1)
    rr = jnp.arange(W2)[:, None]
    cc = jnp.arange(W2)[None, :]
    s_mat = (((cc % 2 == 0) & (rr == cc // 2))
             | ((cc % 2 == 1) & (rr == W2 // 2 + cc // 2))).astype(jnp.bfloat16)

    x1f = x1.reshape(B, Cin, H * W)
    x2f = x2.reshape(B, Cf, H2 * W2)

    out = pl.pallas_call(
        functools.partial(_fused_body, R, W2, Cf, nt),
        out_shape=jax.ShapeDtypeStruct((B, Cf, H2 * W2), jnp.float32),
        grid=(B, nt),
        in_specs=[
            pl.BlockSpec((1, Cin, (R // 2) * W), lambda b, t: (b, 0, t)),
            pl.BlockSpec((1, Cf, TP), lambda b, t: (b, 0, t)),
            pl.BlockSpec((1, Cf, W2),
                         lambda b, t: (b, 0, jnp.maximum(R * t - 1, 0))),
            pl.BlockSpec((1, Cf, W2),
                         lambda b, t: (b, 0, jnp.minimum(R * (t + 1), H2 - 1))),
            pl.BlockSpec((4 * Cf, Cin), lambda b, t: (0, 0)),
            pl.BlockSpec((9, Cf, Cf), lambda b, t: (0, 0, 0)),
            pl.BlockSpec((Cf, Cf), lambda b, t: (0, 0)),
            pl.BlockSpec((W2, W2), lambda b, t: (0, 0)),
            pl.BlockSpec((Cf, 1), lambda b, t: (0, 0)),
            pl.BlockSpec((Cf, 1), lambda b, t: (0, 0)),
        ],
        out_specs=pl.BlockSpec((1, Cf, TP), lambda b, t: (b, 0, t)),
        scratch_shapes=[pltpu.VMEM((Cf, TP + 4 * W2), jnp.bfloat16)],
        compiler_params=pltpu.CompilerParams(
            dimension_semantics=("parallel", "parallel"),
            vmem_limit_bytes=64 * 1024 * 1024),
    )(x1f, x2f, x2f, x2f, ww, wm, wc_med, s_mat, bm, bt)
    return out.reshape(B, Cf, H2, W2)
